# Initial kernel scaffold; baseline (speedup 1.0000x reference)
import jax
import jax.numpy as jnp
from jax.experimental import pallas as pl


def kernel(x):
    raise NotImplementedError("write your pallas kernel here")


# speedup vs baseline: 1.4617x; 1.4617x over previous
import functools

import jax
import jax.numpy as jnp
from jax import lax
from jax.experimental import pallas as pl
from jax.experimental.pallas import tpu as pltpu
from jax.experimental.pallas import tpu_sc as plsc

R = 128
N = 32768
L = 16
NC, NS = 2, 16
NW = NC * NS
RPW = R // NW
VREGS = N // L
NDIG = 256


def _sort_body(x_hbm, out_hbm, key_v, pa_v, pb_v, hist_v):
    wid = lax.axis_index("c") * NS + lax.axis_index("s")
    lanes = lax.iota(jnp.int32, L)
    ones = jnp.full((L,), 1, jnp.int32)

    def do_row(r, _):
        row = wid * RPW + r
        pltpu.sync_copy(x_hbm.at[row], key_v)

        def prep(i, _):
            v = key_v[pl.ds(i * L, L)]
            key_v[pl.ds(i * L, L)] = jnp.where(v < 0, v, v ^ jnp.int32(0x7FFFFFFF))
            pa_v[pl.ds(i * L, L)] = lanes * VREGS + i
            return 0

        lax.fori_loop(0, VREGS, prep, 0)

        for p in range(4):
            shift = 8 * p
            src = pa_v if p % 2 == 0 else pb_v
            dst = pb_v if p % 2 == 0 else pa_v

            def zero(j, _):
                hist_v[pl.ds(j * L, L)] = jnp.zeros((L,), jnp.int32)
                return 0

            lax.fori_loop(0, NDIG, zero, 0)

            def histo(i, _):
                pv = src[pl.ds(i * L, L)]
                k = plsc.load_gather(key_v, [pv])
                d = lax.shift_right_logical(k, shift) & 0xFF
                plsc.addupdate_scatter(hist_v, [d * L + lanes], ones)
                return 0

            lax.fori_loop(0, VREGS, histo, 0)

            def scan(j, carry):
                h = hist_v[pl.ds(j * L, L)]
                inc = plsc.cumsum(h)
                hist_v[pl.ds(j * L, L)] = inc - h + carry
                return carry + jnp.sum(h)

            lax.fori_loop(0, NDIG, scan, jnp.int32(0))

            def permute(i, _):
                pv = src[pl.ds(i * L, L)]
                k = plsc.load_gather(key_v, [pv])
                d = lax.shift_right_logical(k, shift) & 0xFF
                hidx = d * L + lanes
                base = plsc.load_gather(hist_v, [hidx])
                plsc.store_scatter(hist_v, [hidx], base + 1)
                if p < 3:
                    addr = (base & (VREGS - 1)) * L + lax.shift_right_logical(base, 11)
                else:
                    addr = base
                plsc.store_scatter(dst, [addr], pv)
                return 0

            lax.fori_loop(0, VREGS, permute, 0)

        pltpu.sync_copy(pa_v, out_hbm.at[row])
        return 0

    lax.fori_loop(0, RPW, do_row, 0)


_sorter = functools.partial(
    pl.kernel,
    mesh=plsc.VectorSubcoreMesh(core_axis_name="c", subcore_axis_name="s"),
    out_type=jax.ShapeDtypeStruct((R, N), jnp.int32),
    compiler_params=pltpu.CompilerParams(needs_layout_passes=False),
    scratch_types=[
        pltpu.VMEM((N,), jnp.int32),
        pltpu.VMEM((N,), jnp.int32),
        pltpu.VMEM((N,), jnp.int32),
        pltpu.VMEM((NDIG * L,), jnp.int32),
    ],
)(_sort_body)


@jax.jit
def kernel(x):
    inds = _sorter(lax.bitcast_convert_type(x, jnp.int32))
    return inds.astype(jnp.int64)


# speedup vs baseline: 1.5432x; 1.0557x over previous
import functools

import jax
import jax.numpy as jnp
from jax import lax
from jax.experimental import pallas as pl
from jax.experimental.pallas import tpu as pltpu
from jax.experimental.pallas import tpu_sc as plsc

R = 128
N = 32768
L = 16
NC, NS = 2, 16
NW = NC * NS
RPW = R // NW
VREGS = N // L
NDIG = 256
C = 4
CV = VREGS // C


def _sort_body(x_hbm, out_hbm, key_v, pa_v, pb_v, h0_v, h1_v, h2_v, h3_v):
    wid = lax.axis_index("c") * NS + lax.axis_index("s")
    lanes = lax.iota(jnp.int32, L)
    ones = jnp.full((L,), 1, jnp.int32)
    hists = (h0_v, h1_v, h2_v, h3_v)

    def do_row(r, _):
        row = wid * RPW + r
        pltpu.sync_copy(x_hbm.at[row], key_v)

        def prep(i, _):
            v = key_v[pl.ds(i * L, L)]
            key_v[pl.ds(i * L, L)] = jnp.where(v < 0, v, v ^ jnp.int32(0x7FFFFFFF))
            pa_v[pl.ds(i * L, L)] = lanes * VREGS + i
            return 0

        lax.fori_loop(0, VREGS, prep, 0, unroll=8)

        for p in range(4):
            shift = 8 * p
            src = pa_v if p % 2 == 0 else pb_v
            dst = pb_v if p % 2 == 0 else pa_v

            def zero(j, _):
                for h in hists:
                    h[pl.ds(j * L, L)] = jnp.zeros((L,), jnp.int32)
                return 0

            lax.fori_loop(0, NDIG, zero, 0, unroll=4)

            def histo(j, _):
                for c, h in enumerate(hists):
                    pv = src[pl.ds((c * CV + j) * L, L)]
                    k = plsc.load_gather(key_v, [pv])
                    d = lax.shift_right_logical(k, shift) & 0xFF
                    plsc.addupdate_scatter(h, [d * L + lanes], ones)
                return 0

            lax.fori_loop(0, CV, histo, 0, unroll=2)

            def scan(j, carry):
                sl = pl.ds(j * L, L)
                hs = [h[sl] for h in hists]
                t = hs[0] + hs[1] + hs[2] + hs[3]
                inc = plsc.cumsum(t)
                off = inc - t + carry
                for c, h in enumerate(hists):
                    h[sl] = off
                    off = off + hs[c]
                return carry + jnp.sum(t)

            lax.fori_loop(0, NDIG, scan, jnp.int32(0))

            def permute(j, _):
                for c, h in enumerate(hists):
                    pv = src[pl.ds((c * CV + j) * L, L)]
                    k = plsc.load_gather(key_v, [pv])
                    d = lax.shift_right_logical(k, shift) & 0xFF
                    hidx = d * L + lanes
                    base = plsc.load_gather(h, [hidx])
                    plsc.store_scatter(h, [hidx], base + 1)
                    if p < 3:
                        addr = (base & (VREGS - 1)) * L + lax.shift_right_logical(base, 11)
                    else:
                        addr = base
                    plsc.store_scatter(dst, [addr], pv)
                return 0

            lax.fori_loop(0, CV, permute, 0, unroll=2)

        pltpu.sync_copy(pa_v, out_hbm.at[row])
        return 0

    lax.fori_loop(0, RPW, do_row, 0)


_sorter = functools.partial(
    pl.kernel,
    mesh=plsc.VectorSubcoreMesh(core_axis_name="c", subcore_axis_name="s"),
    out_type=jax.ShapeDtypeStruct((R, N), jnp.int32),
    compiler_params=pltpu.CompilerParams(needs_layout_passes=False),
    scratch_types=[
        pltpu.VMEM((N,), jnp.int32),
        pltpu.VMEM((N,), jnp.int32),
        pltpu.VMEM((N,), jnp.int32),
    ] + [pltpu.VMEM((NDIG * L,), jnp.int32)] * C,
)(_sort_body)


@jax.jit
def kernel(x):
    inds = _sorter(lax.bitcast_convert_type(x, jnp.int32))
    return inds.astype(jnp.int64)
